# Initial kernel scaffold; baseline (speedup 1.0000x reference)
import jax
import jax.numpy as jnp
from jax.experimental import pallas as pl


def kernel(x, pre_mqa_scale, post_mqa_scale, pre_moe_scale, post_moe_scale, Wq, Wk, Wv, Wo, Wr, w1, w2):
    raise NotImplementedError("write your pallas kernel here")


# speedup vs baseline: 1.4890x; 1.4890x over previous
import functools
import jax
import jax.numpy as jnp
import numpy as np
from jax.experimental import pallas as pl

S = 2048
D = 1024
H = 16
KV = 2
DH = 64
E = 8
DFF = 2048
EPS = 1e-05

SB = 256
RB = 256


def _rms(v, scale, eps=EPS):
    ms = jnp.mean(v * v, axis=-1, keepdims=True)
    return v * jax.lax.rsqrt(ms + eps) * scale


def _f32(v):
    return v.astype(jnp.float32)


def _bf(v):
    return _f32(v.astype(jnp.bfloat16))


def _segsum(p, Sseg):
    hi = _bf(p)
    r1 = p - hi
    hi2 = _bf(r1)
    hi3 = _bf(r1 - hi2)
    acc = jnp.dot(hi, Sseg, preferred_element_type=jnp.float32)
    acc += jnp.dot(hi2, Sseg, preferred_element_type=jnp.float32)
    acc += jnp.dot(hi3, Sseg, preferred_element_type=jnp.float32)
    return acc


def _attn_body(xb_ref, pres_ref, posts_ref, premoe_ref, Wq_ref, Wk_ref,
               Wv_ref, Wo_ref, Wr_ref, y_ref, xn2_ref, w_ref, mask_ref):
    xb = xb_ref[...]
    xn = _rms(xb, pres_ref[...])
    q = jnp.dot(xn, Wq_ref[...], preferred_element_type=jnp.float32)
    kk = jnp.dot(xn, Wk_ref[...], preferred_element_type=jnp.float32)
    vv = jnp.dot(xn, Wv_ref[...], preferred_element_type=jnp.float32)
    k0, k1 = kk[:, :DH], kk[:, DH:]
    v0, v1 = vv[:, :DH], vv[:, DH:]
    krep0 = jnp.concatenate([k0] * H, axis=1)
    krep1 = jnp.concatenate([k1] * H, axis=1)
    r_i = jax.lax.broadcasted_iota(jnp.int32, (H * DH, H), 0)
    c_i = jax.lax.broadcasted_iota(jnp.int32, (H * DH, H), 1)
    Sseg = ((r_i // DH) == c_i).astype(jnp.float32)
    scale = 1.0 / np.sqrt(DH)
    qb = _bf(q)
    A0 = _segsum(qb * _bf(krep0), Sseg) * scale
    A1 = _segsum(qb * _bf(krep1), Sseg) * scale
    m = jnp.maximum(A0, A1)
    h_i = jax.lax.broadcasted_iota(jnp.int32, (SB, H), 1)
    n0 = jnp.minimum(h_i + 1, 8).astype(jnp.float32)
    n1 = jnp.maximum(h_i - 7, 0).astype(jnp.float32)
    e0 = jnp.exp(A0 - m)
    e1 = jnp.exp(A1 - m)
    z = n0 * e0 + n1 * e1
    w0 = e0 / z
    w1 = e1 / z
    rr = jax.lax.broadcasted_iota(jnp.int32, (H, H * DH), 0)
    cc = jax.lax.broadcasted_iota(jnp.int32, (H, H * DH), 1)
    Eseg = ((cc // DH) == rr).astype(jnp.float32)
    W0 = jnp.dot(_bf(w0), Eseg, preferred_element_type=jnp.float32)
    W1 = jnp.dot(_bf(w1), Eseg, preferred_element_type=jnp.float32)
    vb0 = _bf(jnp.concatenate([v0] * H, axis=1))
    vb1 = _bf(jnp.concatenate([v1] * H, axis=1))
    l_i = jax.lax.broadcasted_iota(jnp.int32, (SB, H * DH), 1) // DH
    n0f = jnp.minimum(l_i + 1, 8).astype(jnp.float32)
    n1f = jnp.maximum(l_i - 7, 0).astype(jnp.float32)
    attn = n0f * (W0 * vb0) + n1f * (W1 * vb1)
    mqa = jnp.dot(attn, Wo_ref[...], preferred_element_type=jnp.float32)
    y = xb + _rms(mqa, posts_ref[...])
    y_ref[...] = y
    xn2 = _rms(y, premoe_ref[...])
    xn2_ref[...] = xn2
    logits = jnp.dot(xn2, Wr_ref[...], preferred_element_type=jnp.float32)
    lm = jnp.max(logits, axis=1, keepdims=True)
    p = jnp.exp(logits - lm)
    p = p / jnp.sum(p, axis=1, keepdims=True)
    e_i = jax.lax.broadcasted_iota(jnp.int32, (SB, E), 1)
    m1 = jnp.max(p, axis=1, keepdims=True)
    i1 = jnp.min(jnp.where(p == m1, e_i, E), axis=1, keepdims=True)
    p2 = jnp.where(e_i == i1, -1.0, p)
    m2 = jnp.max(p2, axis=1, keepdims=True)
    i2 = jnp.min(jnp.where(p2 == m2, e_i, E), axis=1, keepdims=True)
    tv1 = m1 / (m1 + m2)
    tv2 = m2 / (m1 + m2)
    w_all = (jnp.where(e_i == i1, tv1, 0.0) + jnp.where(e_i == i2, tv2, 0.0))
    w_ref[...] = w_all
    mask_ref[...] = (w_all > 0).astype(jnp.float32)


def _moe_body(xn2_ref, w_ref, w1_ref, w2_ref, out_ref):
    e = pl.program_id(0)

    @pl.when(e == 0)
    def _init():
        out_ref[...] = jnp.zeros_like(out_ref)

    lane = jax.lax.broadcasted_iota(jnp.int32, (S, E), 1)
    wcol = jnp.sum(jnp.where(lane == e, w_ref[...], 0.0), axis=1,
                   keepdims=True)
    w1e = w1_ref[0]
    w2e = w2_ref[0]
    for r in range(S // RB):
        sl = pl.ds(r * RB, RB)
        xr = xn2_ref[sl, :].astype(jnp.bfloat16)
        h = jnp.dot(xr, w1e, preferred_element_type=jnp.float32)
        h = h / (1.0 + jnp.exp(-h))
        o = jnp.dot(h.astype(jnp.bfloat16), w2e,
                    preferred_element_type=jnp.float32)
        out_ref[sl, :] += wcol[r * RB:(r + 1) * RB] * o


def _final_body(y_ref, moe_ref, postmoe_ref, out_ref):
    out_ref[...] = y_ref[...] + _rms(moe_ref[...], postmoe_ref[...])


def kernel(x, pre_mqa_scale, post_mqa_scale, pre_moe_scale, post_moe_scale,
           Wq, Wk, Wv, Wo, Wr, w1, w2):
    xs = x.reshape(S, D).astype(jnp.float32)
    pres = pre_mqa_scale.reshape(1, D)
    posts = post_mqa_scale.reshape(1, D)
    premoe = pre_moe_scale.reshape(1, D)
    postmoe = post_moe_scale.reshape(1, D)
    w1b = w1.astype(jnp.bfloat16)
    w2b = w2.astype(jnp.bfloat16)

    full = lambda shape: pl.BlockSpec(shape, lambda i: (0,) * len(shape))
    y, xn2, w_all, mask = pl.pallas_call(
        _attn_body,
        grid=(S // SB,),
        in_specs=[
            pl.BlockSpec((SB, D), lambda i: (i, 0)),
            full((1, D)), full((1, D)), full((1, D)),
            full((D, H * DH)), full((D, KV * DH)), full((D, KV * DH)),
            full((H * DH, D)), full((D, E)),
        ],
        out_specs=[
            pl.BlockSpec((SB, D), lambda i: (i, 0)),
            pl.BlockSpec((SB, D), lambda i: (i, 0)),
            pl.BlockSpec((SB, E), lambda i: (i, 0)),
            pl.BlockSpec((SB, E), lambda i: (i, 0)),
        ],
        out_shape=[
            jax.ShapeDtypeStruct((S, D), jnp.float32),
            jax.ShapeDtypeStruct((S, D), jnp.float32),
            jax.ShapeDtypeStruct((S, E), jnp.float32),
            jax.ShapeDtypeStruct((S, E), jnp.float32),
        ],
    )(xs, pres, posts, premoe, Wq, Wk, Wv, Wo, Wr)

    moe_out = pl.pallas_call(
        _moe_body,
        grid=(E,),
        in_specs=[
            pl.BlockSpec((S, D), lambda e: (0, 0)),
            pl.BlockSpec((S, E), lambda e: (0, 0)),
            pl.BlockSpec((1, D, DFF), lambda e: (e, 0, 0)),
            pl.BlockSpec((1, DFF, D), lambda e: (e, 0, 0)),
        ],
        out_specs=pl.BlockSpec((S, D), lambda e: (0, 0)),
        out_shape=jax.ShapeDtypeStruct((S, D), jnp.float32),
    )(xn2, w_all, w1b, w2b)

    out = pl.pallas_call(
        _final_body,
        grid=(S // SB,),
        in_specs=[
            pl.BlockSpec((SB, D), lambda i: (i, 0)),
            pl.BlockSpec((SB, D), lambda i: (i, 0)),
            full((1, D)),
        ],
        out_specs=pl.BlockSpec((SB, D), lambda i: (i, 0)),
        out_shape=jax.ShapeDtypeStruct((S, D), jnp.float32),
    )(y, moe_out, postmoe)

    return out.reshape(1, S, D), mask
